# baseline (speedup 1.0000x reference)
import numpy as np
import jax
import jax.numpy as jnp
from jax.experimental import pallas as pl
from jax.experimental.pallas import tpu as pltpu

_N = 360

_DN_T = (((1,), (1,)), ((), ()))

_WM1_CHUNKS = 1
_WM2_ROWS = (200,)


def _fused(x_ref, w1_hbm, b1_hbm, w2_hbm, b2_hbm, w3_hbm, b3_hbm,
           bg_hbm, bm1_hbm, bm2_hbm, wg_hbm, wm1_hbm, wm2_hbm,
           out_ref, wg_s, wm1_s, wm2_s,
           w1_s, b1_s, w2_s, b2_s, w3_s, b3_s, bg_s, bm1_s, bm2_s, sems):
    f32 = jnp.float32

    def mm_t(v, w):
        return jax.lax.dot_general(v, w, _DN_T, preferred_element_type=f32)

    cp_g = pltpu.make_async_copy(wg_hbm, wg_s, sems.at[0])
    cp_g.start()
    cp_m1 = [pltpu.make_async_copy(wm1_hbm, wm1_s, sems.at[1])]
    cp_m1[0].start()
    cp_m2 = [pltpu.make_async_copy(wm2_hbm, wm2_s, sems.at[2])]
    cp_m2[0].start()
    smalls = [(w1_hbm, w1_s), (b1_hbm, b1_s), (w2_hbm, w2_s), (b2_hbm, b2_s),
              (w3_hbm, w3_s), (b3_hbm, b3_s), (bg_hbm, bg_s),
              (bm1_hbm, bm1_s), (bm2_hbm, bm2_s)]
    cps = []
    for k, (src, dst) in enumerate(smalls):
        cp = pltpu.make_async_copy(src, dst, sems.at[3 + k])
        cp.start()
        cps.append(cp)
    (cp_w1, cp_b1, cp_w2, cp_b2, cp_w3, cp_b3, cp_bg, cp_bm1, cp_bm2) = cps

    i = jax.lax.broadcasted_iota(jnp.int32, (_N, 1), 0)
    fi = i.astype(f32)

    ang = fi * np.float32(2.0 * np.pi / (_N - 1))
    scan = jnp.transpose(x_ref[0:1, 0:_N])
    nx = scan * jnp.cos(ang)
    ny = scan * jnp.sin(ang)

    end = (i == 0) | (i == (_N - 1))
    dis = jnp.where(end, np.float32(1.0 / np.sqrt(2.0)),
                    np.float32(1.0 / np.sqrt(3.0)))
    cd = dis * dis
    cl = jnp.where(i == 0, 0.0, dis * jnp.roll(dis, 1, axis=0))
    cu = jnp.where(i == (_N - 1), 0.0, dis * jnp.roll(dis, -1, axis=0))

    def agg(v):
        return cd * v + cl * jnp.roll(v, 1, axis=0) + cu * jnp.roll(v, -1, axis=0)

    nodes = jnp.concatenate([nx, ny], axis=1)
    cp_w1.wait()
    cp_b1.wait()
    xw = mm_t(nodes, w1_s[:])
    h = jnp.maximum(agg(xw) + b1_s[:], 0.0)

    cp_w2.wait()
    cp_b2.wait()
    h = jnp.maximum(agg(mm_t(h, w2_s[:])) + b2_s[:], 0.0)
    cp_w3.wait()
    cp_b3.wait()
    h = jnp.maximum(agg(mm_t(h, w3_s[:])) + b3_s[:], 0.0)

    g = jnp.mean(h, axis=0, keepdims=True)
    cp_g.wait()
    cp_bg.wait()
    c = mm_t(g, wg_s[:]) + bg_s[:]
    for cp in cp_m1:
        cp.wait()
    cp_bm1.wait()
    m = jnp.maximum(mm_t(c, wm1_s[:]) + bm1_s[:], 0.0)
    for cp in cp_m2:
        cp.wait()
    cp_bm2.wait()
    row = mm_t(m, wm2_s[:]) + bm2_s[:]
    out_ref[:] = row.reshape(1, 10, 10, 2)


@jax.jit
def _run(x, W1, b1, W2, b2, W3, b3, Wg, bg, Wm1, bm1, Wm2, bm2):
    vmem = pl.BlockSpec(memory_space=pltpu.MemorySpace.VMEM)
    hbm = pl.BlockSpec(memory_space=pltpu.MemorySpace.HBM)
    out = pl.pallas_call(
        _fused,
        out_shape=jax.ShapeDtypeStruct((1, 10, 10, 2), jnp.float32),
        in_specs=[vmem] + [hbm] * 12,
        out_specs=vmem,
        scratch_shapes=[
            pltpu.VMEM((512, 64), jnp.float32),
            pltpu.VMEM((1024, 512), jnp.float32),
            pltpu.VMEM((200, 1024), jnp.float32),
            pltpu.VMEM((64, 2), jnp.float32),
            pltpu.VMEM((64,), jnp.float32),
            pltpu.VMEM((64, 64), jnp.float32),
            pltpu.VMEM((64,), jnp.float32),
            pltpu.VMEM((64, 64), jnp.float32),
            pltpu.VMEM((64,), jnp.float32),
            pltpu.VMEM((512,), jnp.float32),
            pltpu.VMEM((1024,), jnp.float32),
            pltpu.VMEM((200,), jnp.float32),
            pltpu.SemaphoreType.DMA((12,)),
        ],
    )(x, W1, b1, W2, b2, W3, b3, bg, bm1, bm2, Wg, Wm1, Wm2)
    return out


def kernel(x, W1, b1, W2, b2, W3, b3, Wg, bg, Wm1, bm1, Wm2, bm2):
    return _run(x, W1, b1, W2, b2, W3, b3, Wg, bg, Wm1, bm1, Wm2, bm2)


# speedup vs baseline: 1.1227x; 1.1227x over previous
import numpy as np
import jax
import jax.numpy as jnp
from jax.experimental import pallas as pl
from jax.experimental.pallas import tpu as pltpu

_N = 360

_DN_T = (((1,), (1,)), ((), ()))

_WM1_CHUNKS = 1
_WM2_ROWS = (200,)


def _fused(x_ref, w1_ref, b1_ref, w2_ref, b2_ref, w3_ref, b3_ref,
           bg_ref, bm1_ref, bm2_ref, wg_hbm, wm1_hbm, wm2_hbm,
           out_ref, wg_s, wm1_s, wm2_s, sems):
    f32 = jnp.float32

    def mm_t(v, w):
        return jax.lax.dot_general(v, w, _DN_T, preferred_element_type=f32)

    cp_g = pltpu.make_async_copy(wg_hbm, wg_s, sems.at[0])
    cp_g.start()
    cp_m1 = [pltpu.make_async_copy(wm1_hbm, wm1_s, sems.at[1])]
    cp_m1[0].start()
    cp_m2 = [pltpu.make_async_copy(wm2_hbm, wm2_s, sems.at[2])]
    cp_m2[0].start()

    i = jax.lax.broadcasted_iota(jnp.int32, (_N, 1), 0)
    fi = i.astype(f32)

    ang = fi * np.float32(2.0 * np.pi / (_N - 1))
    scan = jnp.transpose(x_ref[0:1, 0:_N])
    nx = scan * jnp.cos(ang)
    ny = scan * jnp.sin(ang)

    end = (i == 0) | (i == (_N - 1))
    dis = jnp.where(end, np.float32(1.0 / np.sqrt(2.0)),
                    np.float32(1.0 / np.sqrt(3.0)))
    cd = dis * dis
    cl = jnp.where(i == 0, 0.0, dis * jnp.roll(dis, 1, axis=0))
    cu = jnp.where(i == (_N - 1), 0.0, dis * jnp.roll(dis, -1, axis=0))

    def agg(v):
        return cd * v + cl * jnp.roll(v, 1, axis=0) + cu * jnp.roll(v, -1, axis=0)

    nodes = jnp.concatenate([nx, ny], axis=1)
    xw = mm_t(nodes, w1_ref[:])
    h = jnp.maximum(agg(xw) + b1_ref[:], 0.0)

    h = jnp.maximum(agg(mm_t(h, w2_ref[:])) + b2_ref[:], 0.0)
    h = jnp.maximum(agg(mm_t(h, w3_ref[:])) + b3_ref[:], 0.0)

    g = jnp.mean(h, axis=0, keepdims=True)
    cp_g.wait()
    c = mm_t(g, wg_s[:]) + bg_ref[:]
    for cp in cp_m1:
        cp.wait()
    m = jnp.maximum(mm_t(c, wm1_s[:]) + bm1_ref[:], 0.0)
    for cp in cp_m2:
        cp.wait()
    row = mm_t(m, wm2_s[:]) + bm2_ref[:]
    out_ref[:] = row.reshape(1, 10, 10, 2)


@jax.jit
def _run(x, W1, b1, W2, b2, W3, b3, Wg, bg, Wm1, bm1, Wm2, bm2):
    vmem = pl.BlockSpec(memory_space=pltpu.MemorySpace.VMEM)
    hbm = pl.BlockSpec(memory_space=pltpu.MemorySpace.HBM)
    out = pl.pallas_call(
        _fused,
        out_shape=jax.ShapeDtypeStruct((1, 10, 10, 2), jnp.float32),
        in_specs=[vmem] * 10 + [hbm] * 3,
        out_specs=vmem,
        scratch_shapes=[
            pltpu.VMEM((512, 64), jnp.float32),
            pltpu.VMEM((1024, 512), jnp.float32),
            pltpu.VMEM((200, 1024), jnp.float32),
            pltpu.SemaphoreType.DMA((1 + _WM1_CHUNKS + len(_WM2_ROWS),)),
        ],
    )(x, W1, b1, W2, b2, W3, b3, bg, bm1, bm2, Wg, Wm1, Wm2)
    return out


def kernel(x, W1, b1, W2, b2, W3, b3, Wg, bg, Wm1, bm1, Wm2, bm2):
    return _run(x, W1, b1, W2, b2, W3, b3, Wg, bg, Wm1, bm1, Wm2, bm2)
